# baseline (speedup 1.0000x reference)
import functools

import jax
import jax.numpy as jnp
from jax import lax
from jax.experimental import pallas as pl
from jax.experimental.pallas import tpu as pltpu
from jax.experimental.pallas import tpu_sc as plsc

N_NODES = 10000
N_EDGES = 160000

NW = 32
NPW = 320
NPAD = NW * NPW
EPW = 5120
EPAD = NW * EPW
CH = 4096
EB = 2048
GRID = EPAD // EB

_MESH = dict(core_axis_name="c", subcore_axis_name="s")


def _wid():
    return lax.axis_index("s") * 2 + lax.axis_index("c")


def _edge_pos_body(px_hbm, py_hbm, src_hbm, dstg_hbm, ep_hbm,
                   idx_v, val_v, rows_v, sem):
    base = _wid() * EPW
    zf = jnp.zeros((16,), jnp.float32)

    def zv(i, _):
        rows_v[pl.ds(i * 16, 16)] = zf
        return 0

    lax.fori_loop(0, 128, zv, 0)
    lane16 = lax.iota(jnp.int32, 16) * 16

    def chunk(i, _):
        b = pl.multiple_of(base + i * 128, 128)
        for col, ids in ((0, src_hbm), (1, src_hbm), (2, dstg_hbm),
                         (3, dstg_hbm)):
            tab = px_hbm if col % 2 == 0 else py_hbm
            pltpu.sync_copy(ids.at[pl.ds(b, 128)], idx_v)
            pltpu.async_copy(tab.at[idx_v], val_v, sem).wait()
            for s in range(8):
                iv = lane16 + (s * 256 + col)
                plsc.store_scatter(rows_v, [iv], val_v[pl.ds(s * 16, 16)])
        pltpu.sync_copy(rows_v, ep_hbm.at[pl.ds(b * 16, 2048)])
        return 0

    lax.fori_loop(0, EPW // 128, chunk, 0)


def _sc_edge_pos(px, py, src, dstg):
    k = pl.kernel(
        _edge_pos_body,
        out_type=jax.ShapeDtypeStruct((EPAD * 16,), jnp.float32),
        mesh=plsc.VectorSubcoreMesh(**_MESH),
        compiler_params=pltpu.CompilerParams(needs_layout_passes=False),
        scratch_types=[
            pltpu.VMEM((128,), jnp.int32),
            pltpu.VMEM((128,), jnp.float32),
            pltpu.VMEM((2048,), jnp.float32),
            pltpu.SemaphoreType.DMA,
        ],
    )
    return k(px, py, src, dstg).reshape(EPAD, 16)


def _segmax_body(nck, h_hbm, dst_hbm, src_hbm, xg_hbm,
                 idb0, mid0, mloc0, rows0, sem0,
                 idb1, mid1, mloc1, rows1, sem1, midx, acc):
    lo = _wid() * NPW
    zf = jnp.zeros((16,), jnp.float32)

    def zrow(r, _):
        for k in range(8):
            acc[r, pl.ds(16 * k, 16)] = zf
        return 0

    lax.fori_loop(0, NPW + 8, zrow, 0)

    lane = lax.iota(jnp.int32, 16)
    padrow = jnp.full((16,), NPW, jnp.int32)
    base_w = _wid() * EPW
    npw_u = jnp.uint32(NPW)
    slots = ((idb0, mid0, mloc0, rows0, sem0),
             (idb1, mid1, mloc1, rows1, sem1))

    def scan_chunk(t, slot):
        idb, mid, mloc, rows, sem = slots[slot]
        cb = pl.multiple_of(t * CH, CH)
        pltpu.sync_copy(dst_hbm.at[pl.ds(cb, CH)], idb)

        def scanvec(i, m):
            v = idb[pl.ds(i * 16, 16)]
            r = v - lo
            mask = plsc.bitcast(r, jnp.uint32) < npw_u
            eid = (cb + i * 16) + lane
            plsc.store_compressed(mid.at[pl.ds(m, 16)], eid, mask=mask)
            plsc.store_compressed(mloc.at[pl.ds(m, 16)], r, mask=mask)
            return m + plsc.all_reduce_population_count(mask)[0]

        m = lax.fori_loop(0, CH // 16, scanvec, 0, unroll=2)

        for k in range(8):
            mid[pl.ds(m + 16 * k, 16)] = base_w + (16 * k) + lane
            mloc[pl.ds(m + 16 * k, 16)] = padrow
        pltpu.make_async_copy(h_hbm.at[mid.at[pl.ds(0, 128)]], rows,
                              sem).start()
        return m

    def rmw_batch(mloc, rows, jb):
        def edge16(i2, _):
            nd16 = mloc[pl.ds(jb + i2 * 16, 16)]
            for l in range(16):
                nd = nd16[l]
                for k in range(nck):
                    s = pl.ds(16 * k, 16)
                    acc[nd, s] = jnp.maximum(acc[nd, s], rows[i2 * 16 + l, s])
            return 0

        lax.fori_loop(0, 8, edge16, 0)

    def process_chunk(slot, m):
        idb, mid, mloc, rows, sem = slots[slot]
        pltpu.make_async_copy(h_hbm.at[mid.at[pl.ds(0, 128)]], rows,
                              sem).wait()
        rmw_batch(mloc, rows, 0)
        nb = (m + 127) // 128

        def proc(j, _):
            jb = pl.multiple_of(j * 128, 128)
            pltpu.async_copy(h_hbm.at[mid.at[pl.ds(jb, 128)]], rows,
                             sem).wait()
            rmw_batch(mloc, rows, jb)
            return 0

        lax.fori_loop(1, nb, proc, 0)

    nchunks = EPAD // CH
    m0 = scan_chunk(0, 0)

    def pair(u, m_even):
        m_odd = scan_chunk(2 * u + 1, 1)
        process_chunk(0, m_even)
        m_next = lax.cond(u < nchunks // 2 - 1,
                          lambda: scan_chunk(2 * u + 2, 0),
                          lambda: 0)
        process_chunk(1, m_odd)
        return m_next

    lax.fori_loop(0, nchunks // 2, pair, m0)

    dump = EPAD + _wid() * 128

    def scat_chunk(t, _):
        idb, mid, mloc, rows, sem = slots[0]
        cb = pl.multiple_of(t * CH, CH)
        pltpu.sync_copy(src_hbm.at[pl.ds(cb, CH)], idb)

        def scanvec(i, m):
            v = idb[pl.ds(i * 16, 16)]
            r = v - lo
            mask = plsc.bitcast(r, jnp.uint32) < npw_u
            eid = (cb + i * 16) + lane
            plsc.store_compressed(mid.at[pl.ds(m, 16)], eid, mask=mask)
            plsc.store_compressed(mloc.at[pl.ds(m, 16)], r, mask=mask)
            return m + plsc.all_reduce_population_count(mask)[0]

        m = lax.fori_loop(0, CH // 16, scanvec, 0, unroll=2)
        for k in range(8):
            mid[pl.ds(m + 16 * k, 16)] = dump + (16 * k) + lane
            mloc[pl.ds(m + 16 * k, 16)] = padrow
        nb = (m + 127) // 128

        def putb(j, _):
            jb = pl.multiple_of(j * 128, 128)

            def build16(i2, _):
                nd16 = mloc[pl.ds(jb + i2 * 16, 16)]
                for l in range(16):
                    nd = nd16[l]
                    for k in range(8):
                        s = pl.ds(16 * k, 16)
                        rows[i2 * 16 + l, s] = acc[nd, s]
                return 0

            lax.fori_loop(0, 8, build16, 0)
            for k in range(8):
                midx[0, pl.ds(16 * k, 16)] = mid[pl.ds(jb + 16 * k, 16)]
            pltpu.async_copy(rows, xg_hbm.at[midx.at[0]], sem).wait()
            return 0

        lax.fori_loop(0, nb, putb, 0)
        return 0

    lax.fori_loop(0, nchunks, scat_chunk, 0)


def _sc_segmax(h, dst, src, nck):
    slot = [
        pltpu.VMEM((CH,), jnp.int32),
        pltpu.VMEM((CH + 144,), jnp.int32),
        pltpu.VMEM((CH + 144,), jnp.int32),
        pltpu.VMEM((128, 128), jnp.float32),
        pltpu.SemaphoreType.DMA,
    ]
    k = pl.kernel(
        functools.partial(_segmax_body, nck),
        out_type=jax.ShapeDtypeStruct((EPAD + NW * 128, 128), jnp.float32),
        mesh=plsc.VectorSubcoreMesh(**_MESH),
        compiler_params=pltpu.CompilerParams(needs_layout_passes=False),
        scratch_types=slot + slot + [
            pltpu.VMEM((1, 128), jnp.int32),
            pltpu.VMEM((NPW + 8, 128), jnp.float32),
        ],
    )
    return k(h, dst, src)


def _mlp_edge_body(has_x, *refs):
    if has_x:
        xg_ref, ep_ref, wax_ref, war_ref, ba_ref, wb_ref, bb_ref, \
            out_ref = refs
    else:
        ep_ref, war_ref, ba_ref, wb_ref, bb_ref, out_ref = refs
    t = jnp.dot(ep_ref[...], war_ref[...], preferred_element_type=jnp.float32, precision=lax.Precision.HIGHEST)
    if has_x:
        t = t + jnp.dot(xg_ref[...], wax_ref[...],
                        preferred_element_type=jnp.float32, precision=lax.Precision.HIGHEST)
    t = jnp.maximum(t + ba_ref[...], 0.0)
    out_ref[...] = jnp.dot(t, wb_ref[...],
                           preferred_element_type=jnp.float32, precision=lax.Precision.HIGHEST) + bb_ref[...]


def _tc_mlp_edges(xg, ep, wax, war, ba, wb, bb):
    full = lambda a: pl.BlockSpec(a.shape, lambda i: (0, 0))
    eb = lambda w: pl.BlockSpec((EB, w), lambda i: (i, 0))
    has_x = xg is not None
    args = ([xg, ep, wax] if has_x else [ep]) + [war, ba, wb, bb]
    specs = ([eb(128), eb(16), full(wax)] if has_x else [eb(16)]) + \
        [full(war), full(ba), full(wb), full(bb)]
    return pl.pallas_call(
        functools.partial(_mlp_edge_body, has_x),
        grid=(GRID,),
        in_specs=specs,
        out_specs=eb(wb.shape[1]),
        out_shape=jax.ShapeDtypeStruct((EPAD, wb.shape[1]), jnp.float32),
        compiler_params=pltpu.CompilerParams(
            dimension_semantics=("arbitrary",)),
    )(*args)


def _l3_body(xg_ref, ep_ref, wax_ref, war_ref, ba_ref, wb_ref, bb_ref,
             l1w_ref, l1b_ref, l2w_ref, l2b_ref, l3w_ref, l3b_ref,
             out_ref, accs):
    i = pl.program_id(0)
    t = (jnp.dot(ep_ref[...], war_ref[...], preferred_element_type=jnp.float32, precision=lax.Precision.HIGHEST)
         + jnp.dot(xg_ref[...], wax_ref[...], preferred_element_type=jnp.float32, precision=lax.Precision.HIGHEST))
    t = jnp.maximum(t + ba_ref[...], 0.0)
    h = jnp.dot(t, wb_ref[...], preferred_element_type=jnp.float32, precision=lax.Precision.HIGHEST) + bb_ref[...]
    row = i * EB + lax.broadcasted_iota(jnp.int32, (EB, 1), 0)
    h = jnp.where(row < N_EDGES, h, 0.0)
    bmax = jnp.max(h, axis=0, keepdims=True)

    @pl.when(i == 0)
    def _():
        accs[0:1, :] = bmax

    @pl.when(i > 0)
    def _():
        accs[0:1, :] = jnp.maximum(accs[0:1, :], bmax)

    @pl.when(i == GRID - 1)
    def _():
        g = jnp.maximum(accs[0:1, :], 0.0)
        g = jnp.maximum(jnp.dot(g, l1w_ref[...],
                                preferred_element_type=jnp.float32, precision=lax.Precision.HIGHEST)
                        + l1b_ref[...], 0.0)
        g = jnp.maximum(jnp.dot(g, l2w_ref[...],
                                preferred_element_type=jnp.float32, precision=lax.Precision.HIGHEST)
                        + l2b_ref[...], 0.0)
        out_ref[...] = (jnp.dot(g, l3w_ref[...],
                                preferred_element_type=jnp.float32, precision=lax.Precision.HIGHEST)
                        + l3b_ref[...])


def _tc_l3_head(xg, ep, wax, war, ba, wb, bb, l1w, l1b, l2w, l2b, l3w, l3b):
    full = lambda a: pl.BlockSpec(a.shape, lambda i: (0, 0))
    eb = lambda w: pl.BlockSpec((EB, w), lambda i: (i, 0))
    return pl.pallas_call(
        _l3_body,
        grid=(GRID,),
        in_specs=[eb(128), eb(16), full(wax), full(war), full(ba), full(wb),
                  full(bb), full(l1w), full(l1b), full(l2w), full(l2b),
                  full(l3w), full(l3b)],
        out_specs=full(jnp.zeros((1, 1))),
        out_shape=jax.ShapeDtypeStruct((1, 1), jnp.float32),
        scratch_shapes=[pltpu.VMEM((8, 256), jnp.float32)],
        compiler_params=pltpu.CompilerParams(
            dimension_semantics=("arbitrary",)),
    )(xg, ep, wax, war, ba, wb, bb, l1w, l1b, l2w, l2b, l3w, l3b)


def _war16(w):
    return jnp.concatenate([w, -w, jnp.zeros((12, w.shape[1]), w.dtype)])


def _padrows(w, n):
    return jnp.pad(w, ((0, n - w.shape[0]), (0, 0)))


def kernel(pos, edge_index, w1a, b1a, w1b, b1b, w2a, b2a, w2b, b2b,
           w3a, b3a, w3b, b3b, l1w, l1b, l2w, l2b, l3w, l3b):
    src = edge_index[0]
    dst = edge_index[1]
    padn = EPAD - N_EDGES
    pad_src = jnp.arange(padn, dtype=jnp.int32) % N_NODES
    src_p = jnp.concatenate([src, pad_src])
    dst_p = jnp.concatenate([dst, jnp.full((padn,), -1, jnp.int32)])
    dst_g = jnp.concatenate([dst, pad_src])
    px = pos[:, 0]
    py = pos[:, 1]

    row = lambda b: b.reshape(1, -1)

    ep = _sc_edge_pos(px, py, src_p, dst_g)

    h1 = _tc_mlp_edges(None, ep, None, _war16(w1a), row(b1a),
                       jnp.pad(w1b, ((0, 0), (0, 64))),
                       row(jnp.pad(b1b, (0, 64))))
    xg1 = _sc_segmax(h1, dst_p, src_p, 4)

    h2 = _tc_mlp_edges(xg1, ep, _padrows(w2a[:64], 128), _war16(w2a[64:]),
                       row(b2a), w2b, row(b2b))
    xg2 = _sc_segmax(h2, dst_p, src_p, 8)

    out = _tc_l3_head(xg2, ep, w3a[:128], _war16(w3a[128:]), row(b3a),
                      w3b, row(b3b), l1w, row(l1b), l2w, row(l2b), l3w,
                      row(l3b))
    return out


# speedup vs baseline: 1.3613x; 1.3613x over previous
import functools

import jax
import jax.numpy as jnp
from jax import lax
from jax.experimental import pallas as pl
from jax.experimental.pallas import tpu as pltpu
from jax.experimental.pallas import tpu_sc as plsc

N_NODES = 10000
N_EDGES = 160000

NW = 32
NPW = 320
NPAD = NW * NPW
EPW = 5120
EPAD = NW * EPW
CH = 8192
EB = 2048
GRID = EPAD // EB

_MESH = dict(core_axis_name="c", subcore_axis_name="s")


def _wid():
    return lax.axis_index("s") * 2 + lax.axis_index("c")


def _edge_pos_body(px_hbm, py_hbm, src_hbm, dstg_hbm, ep_hbm,
                   sidx0, didx0, vsx0, vsy0, vdx0, vdy0, rows0, semg0, semo0,
                   sidx1, didx1, vsx1, vsy1, vdx1, vdy1, rows1, semg1, semo1):
    base = _wid() * EPW
    zf = jnp.zeros((16,), jnp.float32)
    slots = ((sidx0, didx0, (vsx0, vsy0, vdx0, vdy0), rows0, semg0, semo0),
             (sidx1, didx1, (vsx1, vsy1, vdx1, vdy1), rows1, semg1, semo1))

    for sl in range(2):
        rows = slots[sl][3]

        def zv(i, _, rows=rows):
            rows[pl.ds(i * 16, 16)] = zf
            return 0

        lax.fori_loop(0, 128, zv, 0)

    lane16 = lax.iota(jnp.int32, 16) * 16
    nch = EPW // 128

    def gathers(slot):
        sidx, didx, vals, rows, semg, semo = slots[slot]
        return ((px_hbm, sidx, vals[0]), (py_hbm, sidx, vals[1]),
                (px_hbm, didx, vals[2]), (py_hbm, didx, vals[3]))

    def load(t, slot):
        sidx, didx, vals, rows, semg, semo = slots[slot]
        b = pl.multiple_of(base + t * 128, 128)
        pltpu.sync_copy(src_hbm.at[pl.ds(b, 128)], sidx)
        pltpu.sync_copy(dstg_hbm.at[pl.ds(b, 128)], didx)
        for tab, idx, v in gathers(slot):
            pltpu.make_async_copy(tab.at[idx], v, semg).start()

    def flush(t, slot):
        sidx, didx, vals, rows, semg, semo = slots[slot]

        @pl.when(t >= 2)
        def _():
            pltpu.make_async_copy(rows, ep_hbm.at[pl.ds(0, 2048)],
                                  semo).wait()

        for tab, idx, v in gathers(slot):
            pltpu.make_async_copy(tab.at[idx], v, semg).wait()
        for col in range(4):
            v = vals[col]
            for sgrp in range(8):
                iv = lane16 + (sgrp * 256 + col)
                plsc.store_scatter(rows, [iv], v[pl.ds(sgrp * 16, 16)])
        b = pl.multiple_of(base + t * 128, 128)
        pltpu.make_async_copy(rows, ep_hbm.at[pl.ds(b * 16, 2048)],
                              semo).start()

    load(0, 0)
    load(1, 1)

    def pairloop(u, _):
        flush(2 * u, 0)

        @pl.when(2 * u + 2 < nch)
        def _():
            load(2 * u + 2, 0)

        flush(2 * u + 1, 1)

        @pl.when(2 * u + 3 < nch)
        def _():
            load(2 * u + 3, 1)

        return 0

    lax.fori_loop(0, nch // 2, pairloop, 0)
    for sl in range(2):
        rows, semo = slots[sl][3], slots[sl][5]
        pltpu.make_async_copy(rows, ep_hbm.at[pl.ds(0, 2048)], semo).wait()


def _sc_edge_pos(px, py, src, dstg):
    slot = [
        pltpu.VMEM((128,), jnp.int32),
        pltpu.VMEM((128,), jnp.int32),
        pltpu.VMEM((128,), jnp.float32),
        pltpu.VMEM((128,), jnp.float32),
        pltpu.VMEM((128,), jnp.float32),
        pltpu.VMEM((128,), jnp.float32),
        pltpu.VMEM((2048,), jnp.float32),
        pltpu.SemaphoreType.DMA,
        pltpu.SemaphoreType.DMA,
    ]
    k = pl.kernel(
        _edge_pos_body,
        out_type=jax.ShapeDtypeStruct((EPAD * 16,), jnp.float32),
        mesh=plsc.VectorSubcoreMesh(**_MESH),
        compiler_params=pltpu.CompilerParams(needs_layout_passes=False),
        scratch_types=slot + slot,
    )
    return k(px, py, src, dstg).reshape(EPAD, 16)


def _gather_body(tab_hbm, ids_hbm, out_hbm, idx_v, rows_v, sem):
    base = _wid() * EPW

    def chunk(i, _):
        b = pl.multiple_of(base + i * 128, 128)
        pltpu.sync_copy(ids_hbm.at[pl.ds(b, 128)], idx_v)
        pltpu.async_copy(tab_hbm.at[idx_v], rows_v, sem).wait()
        pltpu.sync_copy(rows_v, out_hbm.at[pl.ds(b, 128)])
        return 0

    lax.fori_loop(0, EPW // 128, chunk, 0)


def _sc_gather(tab, ids):
    k = pl.kernel(
        _gather_body,
        out_type=jax.ShapeDtypeStruct((EPAD, 128), jnp.float32),
        mesh=plsc.VectorSubcoreMesh(**_MESH),
        compiler_params=pltpu.CompilerParams(needs_layout_passes=False),
        scratch_types=[
            pltpu.VMEM((128,), jnp.int32),
            pltpu.VMEM((128, 128), jnp.float32),
            pltpu.SemaphoreType.DMA,
        ],
    )
    return k(tab, ids)


def _segmax_body(nck, h_hbm, dst_hbm, out_hbm,
                 idb0, mid0, mloc0, rows0, sem0,
                 idb1, mid1, mloc1, rows1, sem1, acc):
    lo = _wid() * NPW
    zf = jnp.zeros((16,), jnp.float32)

    def zrow(r, _):
        for k in range(8):
            acc[r, pl.ds(16 * k, 16)] = zf
        return 0

    lax.fori_loop(0, NPW + 8, zrow, 0)

    lane = lax.iota(jnp.int32, 16)
    padrow = jnp.full((16,), NPW, jnp.int32)
    base_w = _wid() * EPW
    npw_u = jnp.uint32(NPW)
    slots = ((idb0, mid0, mloc0, rows0, sem0),
             (idb1, mid1, mloc1, rows1, sem1))

    def scan_chunk(t, slot):
        idb, mid, mloc, rows, sem = slots[slot]
        cb = pl.multiple_of(t * CH, CH)
        pltpu.sync_copy(dst_hbm.at[pl.ds(cb, CH)], idb)

        def scanvec(i, m):
            v = idb[pl.ds(i * 16, 16)]
            r = v - lo
            mask = plsc.bitcast(r, jnp.uint32) < npw_u
            eid = (cb + i * 16) + lane
            plsc.store_compressed(mid.at[pl.ds(m, 16)], eid, mask=mask)
            plsc.store_compressed(mloc.at[pl.ds(m, 16)], r, mask=mask)
            return m + plsc.all_reduce_population_count(mask)[0]

        m = lax.fori_loop(0, CH // 16, scanvec, 0, unroll=2)

        for k in range(8):
            mid[pl.ds(m + 16 * k, 16)] = base_w + (16 * k) + lane
            mloc[pl.ds(m + 16 * k, 16)] = padrow
        pltpu.make_async_copy(h_hbm.at[mid.at[pl.ds(0, 128)]], rows,
                              sem).start()
        return m

    def rmw_batch(mloc, rows, jb):
        def edge16(i2, _):
            nd16 = mloc[pl.ds(jb + i2 * 16, 16)]
            for l in range(16):
                nd = nd16[l]
                for k in range(nck):
                    s = pl.ds(16 * k, 16)
                    acc[nd, s] = jnp.maximum(acc[nd, s], rows[i2 * 16 + l, s])
            return 0

        lax.fori_loop(0, 8, edge16, 0)

    def process_chunk(slot, m):
        idb, mid, mloc, rows, sem = slots[slot]
        pltpu.make_async_copy(h_hbm.at[mid.at[pl.ds(0, 128)]], rows,
                              sem).wait()
        rmw_batch(mloc, rows, 0)
        nb = (m + 127) // 128

        def proc(j, _):
            jb = pl.multiple_of(j * 128, 128)
            pltpu.async_copy(h_hbm.at[mid.at[pl.ds(jb, 128)]], rows,
                             sem).wait()
            rmw_batch(mloc, rows, jb)
            return 0

        lax.fori_loop(1, nb, proc, 0)

    nchunks = EPAD // CH
    m0 = scan_chunk(0, 0)

    def pair(u, m_even):
        m_odd = scan_chunk(2 * u + 1, 1)
        process_chunk(0, m_even)
        m_next = lax.cond(u < nchunks // 2 - 1,
                          lambda: scan_chunk(2 * u + 2, 0),
                          lambda: 0)
        process_chunk(1, m_odd)
        return m_next

    lax.fori_loop(0, nchunks // 2, pair, m0)

    lo8 = pl.multiple_of(lo, NPW)
    pltpu.sync_copy(acc.at[pl.ds(0, NPW)], out_hbm.at[pl.ds(lo8, NPW)])


def _sc_segmax(h, dst, nck):
    slot = [
        pltpu.VMEM((CH,), jnp.int32),
        pltpu.VMEM((CH + 144,), jnp.int32),
        pltpu.VMEM((CH + 144,), jnp.int32),
        pltpu.VMEM((128, 128), jnp.float32),
        pltpu.SemaphoreType.DMA,
    ]
    k = pl.kernel(
        functools.partial(_segmax_body, nck),
        out_type=jax.ShapeDtypeStruct((NPAD, 128), jnp.float32),
        mesh=plsc.VectorSubcoreMesh(**_MESH),
        compiler_params=pltpu.CompilerParams(needs_layout_passes=False),
        scratch_types=slot + slot + [
            pltpu.VMEM((NPW + 8, 128), jnp.float32),
        ],
    )
    return k(h, dst)


def _mlp_edge_body(has_x, *refs):
    if has_x:
        xg_ref, ep_ref, wax_ref, war_ref, ba_ref, wb_ref, bb_ref, \
            out_ref = refs
    else:
        ep_ref, war_ref, ba_ref, wb_ref, bb_ref, out_ref = refs
    t = jnp.dot(ep_ref[...], war_ref[...], preferred_element_type=jnp.float32, precision=lax.Precision.HIGHEST)
    if has_x:
        t = t + jnp.dot(xg_ref[...], wax_ref[...],
                        preferred_element_type=jnp.float32, precision=lax.Precision.HIGHEST)
    t = jnp.maximum(t + ba_ref[...], 0.0)
    out_ref[...] = jnp.dot(t, wb_ref[...],
                           preferred_element_type=jnp.float32, precision=lax.Precision.HIGHEST) + bb_ref[...]


def _tc_mlp_edges(xg, ep, wax, war, ba, wb, bb):
    full = lambda a: pl.BlockSpec(a.shape, lambda i: (0, 0))
    eb = lambda w: pl.BlockSpec((EB, w), lambda i: (i, 0))
    has_x = xg is not None
    args = ([xg, ep, wax] if has_x else [ep]) + [war, ba, wb, bb]
    specs = ([eb(128), eb(16), full(wax)] if has_x else [eb(16)]) + \
        [full(war), full(ba), full(wb), full(bb)]
    return pl.pallas_call(
        functools.partial(_mlp_edge_body, has_x),
        grid=(GRID,),
        in_specs=specs,
        out_specs=eb(wb.shape[1]),
        out_shape=jax.ShapeDtypeStruct((EPAD, wb.shape[1]), jnp.float32),
        compiler_params=pltpu.CompilerParams(
            dimension_semantics=("arbitrary",)),
    )(*args)


def _l3_body(xg_ref, ep_ref, wax_ref, war_ref, ba_ref, wb_ref, bb_ref,
             l1w_ref, l1b_ref, l2w_ref, l2b_ref, l3w_ref, l3b_ref,
             out_ref, accs):
    i = pl.program_id(0)
    t = (jnp.dot(ep_ref[...], war_ref[...], preferred_element_type=jnp.float32, precision=lax.Precision.HIGHEST)
         + jnp.dot(xg_ref[...], wax_ref[...], preferred_element_type=jnp.float32, precision=lax.Precision.HIGHEST))
    t = jnp.maximum(t + ba_ref[...], 0.0)
    h = jnp.dot(t, wb_ref[...], preferred_element_type=jnp.float32, precision=lax.Precision.HIGHEST) + bb_ref[...]
    row = i * EB + lax.broadcasted_iota(jnp.int32, (EB, 1), 0)
    h = jnp.where(row < N_EDGES, h, 0.0)
    bmax = jnp.max(h, axis=0, keepdims=True)

    @pl.when(i == 0)
    def _():
        accs[0:1, :] = bmax

    @pl.when(i > 0)
    def _():
        accs[0:1, :] = jnp.maximum(accs[0:1, :], bmax)

    @pl.when(i == GRID - 1)
    def _():
        g = jnp.maximum(accs[0:1, :], 0.0)
        g = jnp.maximum(jnp.dot(g, l1w_ref[...],
                                preferred_element_type=jnp.float32, precision=lax.Precision.HIGHEST)
                        + l1b_ref[...], 0.0)
        g = jnp.maximum(jnp.dot(g, l2w_ref[...],
                                preferred_element_type=jnp.float32, precision=lax.Precision.HIGHEST)
                        + l2b_ref[...], 0.0)
        out_ref[...] = (jnp.dot(g, l3w_ref[...],
                                preferred_element_type=jnp.float32, precision=lax.Precision.HIGHEST)
                        + l3b_ref[...])


def _tc_l3_head(xg, ep, wax, war, ba, wb, bb, l1w, l1b, l2w, l2b, l3w, l3b):
    full = lambda a: pl.BlockSpec(a.shape, lambda i: (0, 0))
    eb = lambda w: pl.BlockSpec((EB, w), lambda i: (i, 0))
    return pl.pallas_call(
        _l3_body,
        grid=(GRID,),
        in_specs=[eb(128), eb(16), full(wax), full(war), full(ba), full(wb),
                  full(bb), full(l1w), full(l1b), full(l2w), full(l2b),
                  full(l3w), full(l3b)],
        out_specs=full(jnp.zeros((1, 1))),
        out_shape=jax.ShapeDtypeStruct((1, 1), jnp.float32),
        scratch_shapes=[pltpu.VMEM((8, 256), jnp.float32)],
        compiler_params=pltpu.CompilerParams(
            dimension_semantics=("arbitrary",)),
    )(xg, ep, wax, war, ba, wb, bb, l1w, l1b, l2w, l2b, l3w, l3b)


def _war16(w):
    return jnp.concatenate([w, -w, jnp.zeros((12, w.shape[1]), w.dtype)])


def _padrows(w, n):
    return jnp.pad(w, ((0, n - w.shape[0]), (0, 0)))


def kernel(pos, edge_index, w1a, b1a, w1b, b1b, w2a, b2a, w2b, b2b,
           w3a, b3a, w3b, b3b, l1w, l1b, l2w, l2b, l3w, l3b):
    src = edge_index[0]
    dst = edge_index[1]
    padn = EPAD - N_EDGES
    pad_src = jnp.arange(padn, dtype=jnp.int32) % N_NODES
    src_p = jnp.concatenate([src, pad_src])
    dst_p = jnp.concatenate([dst, jnp.full((padn,), -1, jnp.int32)])
    dst_g = jnp.concatenate([dst, pad_src])
    px = pos[:, 0]
    py = pos[:, 1]

    row = lambda b: b.reshape(1, -1)

    ep = _sc_edge_pos(px, py, src_p, dst_g)

    h1 = _tc_mlp_edges(None, ep, None, _war16(w1a), row(b1a),
                       jnp.pad(w1b, ((0, 0), (0, 64))),
                       row(jnp.pad(b1b, (0, 64))))
    x1 = _sc_segmax(h1, dst_p, 4)

    xg1 = _sc_gather(x1, src_p)
    h2 = _tc_mlp_edges(xg1, ep, _padrows(w2a[:64], 128), _war16(w2a[64:]),
                       row(b2a), w2b, row(b2b))
    x2 = _sc_segmax(h2, dst_p, 8)

    xg2 = _sc_gather(x2, src_p)
    out = _tc_l3_head(xg2, ep, w3a[:128], _war16(w3a[128:]), row(b3a),
                      w3b, row(b3b), l1w, row(l1b), l2w, row(l2b), l3w,
                      row(l3b))
    return out
